# baseline (speedup 1.0000x reference)
import functools

import jax
import jax.numpy as jnp
from jax.experimental import pallas as pl

H = 16
L = 3


def _gru_prep(p):
    grus = [p['na'], p['ta'], p['aa']]
    Wih = []
    Whh = []
    for l in range(L):
        rows_ih = []
        rows_hh = []
        for gate in range(3):
            for gi, g in enumerate(grus):
                w_ih = (g['Wih0'] if l == 0 else g['Wih' + str(l)])
                rows_ih.append((gi, w_ih[gate * H:(gate + 1) * H]))
                rows_hh.append((gi, g['Whh'][l][gate * H:(gate + 1) * H]))
        if l == 0:
            Wih.append(jnp.concatenate([w for _, w in rows_ih], axis=0))
        else:
            blocks = []
            for gi, w in rows_ih:
                row = [jnp.zeros((H, H), jnp.float32)] * 3
                row[gi] = w
                blocks.append(jnp.concatenate(row, axis=1))
            Wih.append(jnp.concatenate(blocks, axis=0))
        blocks = []
        for gi, w in rows_hh:
            row = [jnp.zeros((H, H), jnp.float32)] * 3
            row[gi] = w
            blocks.append(jnp.concatenate(row, axis=1))
        Whh.append(jnp.concatenate(blocks, axis=0))
    return Wih, Whh


def _fused_gru(x, Wih, Whh):
    n = x.shape[0]
    x_t = jnp.transpose(x, (1, 0, 2))
    inp = x_t
    finals = []
    for l in range(L):
        W = Wih[l]
        U = Whh[l]

        def step(h, x_i, W=W, U=U):
            gi = x_i @ W.T
            gh = h @ U.T
            r = jax.nn.sigmoid(gi[:, 0:48] + gh[:, 0:48])
            z = jax.nn.sigmoid(gi[:, 48:96] + gh[:, 48:96])
            c = jnp.tanh(gi[:, 96:144] + r * gh[:, 96:144])
            h_new = (1.0 - z) * c + z * h
            return h_new, h_new

        h0 = jnp.zeros((n, 48), x.dtype)
        hT, outs = jax.lax.scan(step, h0, inp)
        finals.append(hT)
        inp = outs
    return finals


def _mlp_body(u_ref, w1_ref, b1_ref, w2_ref, b2_ref, o_ref):
    lin = jnp.maximum(u_ref[...] @ w1_ref[...] + b1_ref[...], 0.0)
    o_ref[...] = jax.nn.sigmoid(lin @ w2_ref[...] + b2_ref[...])


def _final_mlp(union, W1, b1, W2, b2):
    Q = union.shape[0]
    BQ = 2048
    return pl.pallas_call(
        _mlp_body,
        grid=(Q // BQ,),
        in_specs=[
            pl.BlockSpec((BQ, 160), lambda i: (i, 0)),
            pl.BlockSpec((160, 256), lambda i: (0, 0)),
            pl.BlockSpec((1, 256), lambda i: (0, 0)),
            pl.BlockSpec((256, 1), lambda i: (0, 0)),
            pl.BlockSpec((1, 1), lambda i: (0, 0)),
        ],
        out_specs=pl.BlockSpec((BQ, 1), lambda i: (i, 0)),
        out_shape=jax.ShapeDtypeStruct((Q, 1), jnp.float32),
    )(union, W1, b1.reshape(1, 256), W2, b2.reshape(1, 1))


def kernel(x, edge_index, q_from, q_to, params):
    n = x.shape[0]
    src, dst = edge_index[0], edge_index[1]

    Wih, Whh = _gru_prep(params)
    finals = _fused_gru(x, Wih, Whh)
    traj_feat = jnp.concatenate([f[:, 0:16] for f in finals], axis=1)
    traj_feat2 = finals[-1][:, 16:32]
    att_feat = finals[-1][:, 32:48]

    A = jnp.concatenate([params['l1_attW'][i][:16] for i in range(4)], axis=1)
    B = jnp.concatenate([params['l1_attW'][i][16:] for i in range(4)], axis=1)
    a_src = att_feat @ A
    a_dst = att_feat @ B + params['l1_attb'][:, 0][None, :]
    Wf = jnp.concatenate([params['l1_fcW'][i][:48] for i in range(4)], axis=1)
    Wt = jnp.concatenate([params['l1_fcW'][i][48:] for i in range(4)], axis=1)
    g = traj_feat @ Wf
    tb = traj_feat @ Wt

    e1 = jax.nn.sigmoid(a_src[src] + a_dst[dst])
    contrib = (e1[:, :, None] * g[src].reshape(-1, 4, 16)).reshape(-1, 64)
    agg1 = jnp.zeros((n, 64), jnp.float32).at[dst].add(contrib)
    h = jax.nn.elu(agg1 + tb)

    s2 = h @ params['l2_attW'][:64]
    d2 = h @ params['l2_attW'][64:]
    u = h @ params['l2_fcW'][:64]
    hb = h @ params['l2_fcW'][64:]
    e2 = jax.nn.sigmoid(s2[src] + d2[dst])
    agg2 = jnp.zeros((n, 64), jnp.float32).at[dst].add(e2 * u[src])
    g_feat = agg2 + hb

    union = jnp.concatenate(
        [g_feat[q_from], g_feat[q_to], traj_feat2[q_from], traj_feat2[q_to]],
        axis=1)
    return _final_mlp(union, params['p_W1'], params['p_b1'],
                      params['p_W2'], params['p_b2'])


# speedup vs baseline: 1.3328x; 1.3328x over previous
import functools

import jax
import jax.numpy as jnp
from jax import lax
from jax.experimental import pallas as pl
from jax.experimental.pallas import tpu as pltpu
from jax.experimental.pallas import tpu_sc as plsc

H = 16
L = 3


def _gru_prep(p):
    grus = [p['na'], p['ta'], p['aa']]
    Wih = []
    Whh = []
    for l in range(L):
        rows_ih = []
        rows_hh = []
        for gate in range(3):
            for gi, g in enumerate(grus):
                w_ih = (g['Wih0'] if l == 0 else g['Wih' + str(l)])
                rows_ih.append((gi, w_ih[gate * H:(gate + 1) * H]))
                rows_hh.append((gi, g['Whh'][l][gate * H:(gate + 1) * H]))
        if l == 0:
            Wih.append(jnp.concatenate([w for _, w in rows_ih], axis=0))
        else:
            blocks = []
            for gi, w in rows_ih:
                row = [jnp.zeros((H, H), jnp.float32)] * 3
                row[gi] = w
                blocks.append(jnp.concatenate(row, axis=1))
            Wih.append(jnp.concatenate(blocks, axis=0))
        blocks = []
        for gi, w in rows_hh:
            row = [jnp.zeros((H, H), jnp.float32)] * 3
            row[gi] = w
            blocks.append(jnp.concatenate(row, axis=1))
        Whh.append(jnp.concatenate(blocks, axis=0))
    return Wih, Whh


def _fused_gru(x, Wih, Whh):
    n = x.shape[0]
    x_t = jnp.transpose(x, (1, 0, 2))
    inp = x_t
    finals = []
    for l in range(L):
        W = Wih[l]
        U = Whh[l]

        def step(h, x_i, W=W, U=U):
            gi = x_i @ W.T
            gh = h @ U.T
            r = jax.nn.sigmoid(gi[:, 0:48] + gh[:, 0:48])
            z = jax.nn.sigmoid(gi[:, 48:96] + gh[:, 48:96])
            c = jnp.tanh(gi[:, 96:144] + r * gh[:, 96:144])
            h_new = (1.0 - z) * c + z * h
            return h_new, h_new

        h0 = jnp.zeros((n, 48), x.dtype)
        hT, outs = jax.lax.scan(step, h0, inp)
        finals.append(hT)
        inp = outs
    return finals


def _gru_body(x_ref, wih0_ref, wih1_ref, wih2_ref, whh_ref,
              at_ref, bt_ref, ab_ref, wft_ref, wtt_ref,
              as_ref, ad_ref, g_ref, tb_ref, tf2_ref,
              s0_ref, s1_ref):
    T = x_ref.shape[0]
    B = x_ref.shape[2]

    def run_layer(l, wih, inp_ref, out_ref):
        U = whh_ref[l]

        def step(t, h):
            xt = inp_ref[t]
            gi = jnp.dot(wih, xt, preferred_element_type=jnp.float32)
            gh = jnp.dot(U, h, preferred_element_type=jnp.float32)
            r = jax.nn.sigmoid(gi[0:48] + gh[0:48])
            z = jax.nn.sigmoid(gi[48:96] + gh[48:96])
            c = jnp.tanh(gi[96:144] + r * gh[96:144])
            h = (1.0 - z) * c + z * h
            if out_ref is not None:
                out_ref[t] = h
            return h

        h0 = jnp.zeros((48, B), jnp.float32)
        return jax.lax.fori_loop(0, T, step, h0)

    f0 = run_layer(0, wih0_ref[...], x_ref, s0_ref)
    f1 = run_layer(1, wih1_ref[...], s0_ref, s1_ref)
    f2 = run_layer(2, wih2_ref[...], s1_ref, None)

    trajT = jnp.concatenate([f0[0:16], f1[0:16], f2[0:16]], axis=0)
    attT = f2[32:48]
    tf2_ref[...] = f2[16:32]
    as_ref[...] = jnp.dot(at_ref[...], attT, preferred_element_type=jnp.float32)
    ad_ref[...] = jnp.dot(bt_ref[...], attT,
                          preferred_element_type=jnp.float32) + ab_ref[...]
    g_ref[...] = jnp.dot(wft_ref[...], trajT, preferred_element_type=jnp.float32)
    tb_ref[...] = jnp.dot(wtt_ref[...], trajT, preferred_element_type=jnp.float32)


def _gru_phase(x, params, block=2048):
    n, T, _ = x.shape
    Wih, Whh = _gru_prep(params)
    A = jnp.concatenate([params['l1_attW'][i][:16] for i in range(4)], axis=1)
    B_ = jnp.concatenate([params['l1_attW'][i][16:] for i in range(4)], axis=1)
    ab = params['l1_attb'][:, 0].reshape(4, 1)
    Wf = jnp.concatenate([params['l1_fcW'][i][:48] for i in range(4)], axis=1)
    Wt = jnp.concatenate([params['l1_fcW'][i][48:] for i in range(4)], axis=1)

    xf = jnp.transpose(x, (1, 2, 0))
    wspec = lambda s: pl.BlockSpec(s, lambda i: tuple(0 for _ in s))
    outs = pl.pallas_call(
        _gru_body,
        grid=(n // block,),
        in_specs=[
            pl.BlockSpec((T, 4, block), lambda i: (0, 0, i)),
            wspec((144, 4)), wspec((144, 48)), wspec((144, 48)),
            wspec((L, 144, 48)),
            wspec((4, 16)), wspec((4, 16)), wspec((4, 1)),
            wspec((64, 48)), wspec((64, 48)),
        ],
        out_specs=[
            pl.BlockSpec((4, block), lambda i: (0, i)),
            pl.BlockSpec((4, block), lambda i: (0, i)),
            pl.BlockSpec((64, block), lambda i: (0, i)),
            pl.BlockSpec((64, block), lambda i: (0, i)),
            pl.BlockSpec((16, block), lambda i: (0, i)),
        ],
        out_shape=[
            jax.ShapeDtypeStruct((4, n), jnp.float32),
            jax.ShapeDtypeStruct((4, n), jnp.float32),
            jax.ShapeDtypeStruct((64, n), jnp.float32),
            jax.ShapeDtypeStruct((64, n), jnp.float32),
            jax.ShapeDtypeStruct((16, n), jnp.float32),
        ],
        scratch_shapes=[
            pltpu.VMEM((T, 48, block), jnp.float32),
            pltpu.VMEM((T, 48, block), jnp.float32),
        ],
    )(xf, Wih[0], Wih[1], Wih[2], jnp.stack(Whh),
      A.T, B_.T, ab, Wf.T, Wt.T)
    return outs


_NR = 25600
_N_PAD = 4 * _NR
_EC = 48


def _sc_edge_agg(srcp, dstp, pay, adt, gh):
    ep = srcp.shape[0]
    e_tile = ep // 16
    n_chunks = e_tile // _EC
    assert n_chunks * _EC == e_tile
    nrh = _NR // 2
    rows_t = nrh // 16
    mesh = plsc.VectorSubcoreMesh(core_axis_name="c", subcore_axis_name="s")

    dnums = lax.GatherDimensionNumbers(
        offset_dims=(), collapsed_slice_dims=(0,), start_index_map=(0,))

    def vperm(vec, idx_vec):
        return lax.gather(vec, idx_vec.reshape(16, 1), dnums,
                          slice_sizes=(1,),
                          mode=lax.GatherScatterMode.PROMISE_IN_BOUNDS)

    @functools.partial(
        pl.kernel,
        mesh=mesh,
        out_type=jax.ShapeDtypeStruct((4 * nrh, 128), jnp.float32),
        scratch_types=[
            pltpu.VMEM((_EC,), jnp.int32),
            pltpu.VMEM((_EC,), jnp.int32),
            pltpu.VMEM((_EC,), jnp.int32),
            pltpu.VMEM((_EC + 16,), jnp.int32),
            pltpu.VMEM((40,), jnp.int32),
            pltpu.VMEM((_EC, 128), jnp.float32),
            pltpu.VMEM((_EC, 128), jnp.float32),
            pltpu.VMEM((_EC, 128), jnp.float32),
            pltpu.VMEM_SHARED((nrh + 8, 128), jnp.float32),
            pltpu.SemaphoreType.DMA,
        ],
    )
    def k(src_hbm, dst_hbm, pay_hbm, ad_hbm, out_hbm,
          sidx, didx, dloc, dlocp, ridx, payv, adv, contrib, acc, sem):
        c = lax.axis_index("c")
        s = lax.axis_index("s")
        tile_e0 = s * e_tile
        zf = (lax.iota(jnp.int32, 16) * 0).astype(jnp.float32)

        def ridx_set(lo):
            i16 = lax.iota(jnp.int32, 16)
            ridx[pl.ds(0, 16)] = i16 + lo
            ridx[pl.ds(16, 16)] = i16 + lo + 16
            ridx[pl.ds(24, 16)] = i16 + lo + 24

        for r in range(2):
            rid = 2 * c + r
            base = rid * _NR

            def zc(j, _):
                for b in range(8):
                    contrib[j, pl.ds(16 * b, 16)] = zf
                return 0
            lax.fori_loop(0, _EC, zc, 0)

            def zcp(k2, _):
                ridx_set(s * rows_t + k2 * 40)
                pltpu.sync_copy(contrib.at[pl.ds(0, 40)], acc.at[ridx])
                return 0
            lax.fori_loop(0, rows_t // 40, zcp, 0)
            plsc.subcore_barrier()

            def chunk(cc, _):
                e0 = tile_e0 + cc * _EC
                pltpu.sync_copy(src_hbm.at[pl.ds(e0, _EC)], sidx)
                pltpu.sync_copy(dst_hbm.at[pl.ds(e0, _EC)], didx)

                def dl(kk, _):
                    d = didx[pl.ds(16 * kk, 16)]
                    loc = d - base
                    m = (loc >= 0) & (loc < _NR)
                    locc = jnp.where(m, loc, _NR)
                    dloc[pl.ds(16 * kk, 16)] = locc >> 1
                    dlocp[pl.ds(16 * kk, 16)] = locc
                    return 0
                lax.fori_loop(0, _EC // 16, dl, 0)

                cp1 = pltpu.async_copy(pay_hbm.at[sidx], payv, sem)
                cp2 = pltpu.async_copy(ad_hbm.at[didx], adv, sem)
                cp1.wait()
                cp2.wait()

                def ce(j, _):
                    loc = dlocp[pl.ds(j, 16)][0]

                    @pl.when(loc < _NR)
                    def _():
                        ssum = payv[j, pl.ds(0, 16)] + adv[j, pl.ds(0, 16)]
                        ev = 1.0 / (1.0 + jnp.exp(-ssum))
                        z16 = lax.iota(jnp.int32, 16) * 0
                        even = (loc & 1) == 0
                        for b in range(4):
                            gi = b if gh == 4 else 0
                            bg = vperm(ev, z16 + gi)
                            v = bg * payv[j, pl.ds(16 + 16 * b, 16)]
                            contrib[j, pl.ds(16 * b, 16)] = (
                                jnp.where(even, v, zf))
                            contrib[j, pl.ds(64 + 16 * b, 16)] = (
                                jnp.where(even, zf, v))
                    return 0
                lax.fori_loop(0, _EC, ce, 0)

                pltpu.sync_copy(contrib, acc.at[dloc], add=True)
                return 0
            lax.fori_loop(0, n_chunks, chunk, 0)
            plsc.subcore_barrier()

            def ocp(k2, _):
                lo = s * rows_t + k2 * 40
                ridx_set(lo)
                pltpu.async_copy(acc.at[ridx],
                                 contrib.at[pl.ds(0, 40)], sem).wait()
                pltpu.sync_copy(contrib.at[pl.ds(0, 40)],
                                out_hbm.at[pl.ds(rid * nrh + lo, 40)])
                return 0
            lax.fori_loop(0, rows_t // 40, ocp, 0)
            plsc.subcore_barrier()

    return k(srcp, dstp, pay, adt)


def _mlp_body(u_ref, w1_ref, b1_ref, w2_ref, b2_ref, o_ref):
    lin = jnp.maximum(u_ref[...] @ w1_ref[...] + b1_ref[...], 0.0)
    o_ref[...] = jax.nn.sigmoid(lin @ w2_ref[...] + b2_ref[...])


def _final_mlp(union, W1, b1, W2, b2):
    Q = union.shape[0]
    BQ = 2048
    return pl.pallas_call(
        _mlp_body,
        grid=(Q // BQ,),
        in_specs=[
            pl.BlockSpec((BQ, 160), lambda i: (i, 0)),
            pl.BlockSpec((160, 256), lambda i: (0, 0)),
            pl.BlockSpec((1, 256), lambda i: (0, 0)),
            pl.BlockSpec((256, 1), lambda i: (0, 0)),
            pl.BlockSpec((1, 1), lambda i: (0, 0)),
        ],
        out_specs=pl.BlockSpec((BQ, 1), lambda i: (i, 0)),
        out_shape=jax.ShapeDtypeStruct((Q, 1), jnp.float32),
    )(union, W1, b1.reshape(1, 256), W2, b2.reshape(1, 1))


def kernel(x, edge_index, q_from, q_to, params):
    n = x.shape[0]
    src, dst = edge_index[0], edge_index[1]

    asT, adT, gT, tbT, tf2T = _gru_phase(x, params)
    a_src, a_dst, g, tb, traj_feat2 = asT.T, adT.T, gT.T, tbT.T, tf2T.T

    e = src.shape[0]
    ep = 16 * _EC * -(-e // (16 * _EC))
    srcp = jnp.concatenate([src, jnp.zeros((ep - e,), src.dtype)])
    dstp = jnp.concatenate([dst, jnp.full((ep - e,), n, dst.dtype)])
    order = jnp.argsort(dstp)
    spread = lambda v: v[order].reshape(_EC, ep // _EC).T.reshape(-1)
    srcp = spread(srcp)
    dstp = spread(dstp)

    z12 = jnp.zeros((n, 12), jnp.float32)
    z48 = jnp.zeros((n, 48), jnp.float32)
    pay1 = jnp.concatenate([a_src, z12, g, z48], axis=1)
    adt1 = jnp.pad(a_dst, ((0, 8), (0, 124)))
    agg1 = _sc_edge_agg(srcp, dstp, pay1, adt1, gh=4).reshape(-1, 64)[:n]
    h = jax.nn.elu(agg1 + tb)

    s2 = h @ params['l2_attW'][:64]
    d2 = h @ params['l2_attW'][64:]
    u = h @ params['l2_fcW'][:64]
    hb = h @ params['l2_fcW'][64:]
    pay2 = jnp.concatenate([s2, jnp.zeros((n, 15), jnp.float32), u, z48],
                           axis=1)
    adt2 = jnp.pad(d2, ((0, 8), (0, 127)))
    agg2 = _sc_edge_agg(srcp, dstp, pay2, adt2, gh=1).reshape(-1, 64)[:n]
    g_feat = agg2 + hb

    union = jnp.concatenate(
        [g_feat[q_from], g_feat[q_to], traj_feat2[q_from], traj_feat2[q_to]],
        axis=1)
    return _final_mlp(union, params['p_W1'], params['p_b1'],
                      params['p_W2'], params['p_b2'])
